# baseline (speedup 1.0000x reference)
import functools

import jax
import jax.numpy as jnp
from jax.experimental import pallas as pl
from jax.experimental.pallas import tpu as pltpu

_NUM_TOP = 300
_NUM_Q = 900
_NUM_C = 80
_BLK = 16


def _detr_kernel(logits_ref, boxes_ref, size4_ref, out_ref,
                 s_ref, bx_ref, rowmax_ref):
    sig = jax.nn.sigmoid(logits_ref[...])
    s_ref[...] = sig
    rowmax_ref[...] = jnp.max(sig, axis=2)

    b = boxes_ref[...]
    xy = b[..., :2] - b[..., 2:] * 0.5
    bx_ref[...] = jnp.concatenate([xy, b[..., 2:]], axis=2) * size4_ref[0][None, None, :]

    iota_q = jax.lax.broadcasted_iota(jnp.int32, (_BLK, _NUM_Q), 1)
    iota_c = jax.lax.broadcasted_iota(jnp.int32, (1, _NUM_C), 1)

    def step(k, carry):
        rm = rowmax_ref[...]
        m = jnp.max(rm, axis=1, keepdims=True)
        qsel = jnp.min(jnp.where(rm == m, iota_q, _NUM_Q), axis=1)
        new_maxes = []
        for i in range(_BLK):
            qi = qsel[i]
            mi = m[i, 0]
            row = s_ref[i, pl.ds(qi, 1), :]
            ci = jnp.min(jnp.where(row == mi, iota_c, _NUM_C))
            new_row = jnp.where(iota_c == ci, -1.0, row)
            s_ref[i, pl.ds(qi, 1), :] = new_row
            new_maxes.append(jnp.max(new_row))
            box4 = bx_ref[i, pl.ds(qi, 1), :]
            lab = jnp.reshape(ci.astype(jnp.float32), (1, 1))
            sv = jnp.reshape(mi, (1, 1))
            row6 = jnp.concatenate([lab, sv, box4], axis=1)
            row6 = jnp.where(mi > 0.0, row6, jnp.zeros_like(row6))
            out_ref[i, pl.ds(k, 1), :] = row6
        nm = jnp.stack(new_maxes)
        rowmax_ref[...] = jnp.where(iota_q == qsel[:, None], nm[:, None], rm)
        return carry

    jax.lax.fori_loop(0, _NUM_TOP, step, 0)


@jax.jit
def kernel(logits, boxes, original_sizes):
    n = logits.shape[0]
    osz = original_sizes[0].astype(jnp.float32)
    size4 = jnp.stack([osz[1], osz[0], osz[1], osz[0]])[None, :]

    grid = (n // _BLK,)
    out = pl.pallas_call(
        _detr_kernel,
        grid=grid,
        in_specs=[
            pl.BlockSpec((_BLK, _NUM_Q, _NUM_C), lambda b: (b, 0, 0)),
            pl.BlockSpec((_BLK, _NUM_Q, 4), lambda b: (b, 0, 0)),
            pl.BlockSpec((1, 4), lambda b: (0, 0)),
        ],
        out_specs=pl.BlockSpec((_BLK, _NUM_TOP, 6), lambda b: (b, 0, 0)),
        out_shape=jax.ShapeDtypeStruct((n, _NUM_TOP, 6), jnp.float32),
        scratch_shapes=[
            pltpu.VMEM((_BLK, _NUM_Q, _NUM_C), jnp.float32),
            pltpu.VMEM((_BLK, _NUM_Q, 4), jnp.float32),
            pltpu.VMEM((_BLK, _NUM_Q), jnp.float32),
        ],
    )(logits, boxes, size4)
    return out


# speedup vs baseline: 9.1511x; 9.1511x over previous
import functools

import jax
import jax.numpy as jnp
from jax.experimental import pallas as pl
from jax.experimental.pallas import tpu as pltpu

_NUM_TOP = 300
_NUM_Q = 900
_NUM_C = 80
_BLK = 16


def _detr_kernel(logits_ref, boxes_ref, size4_ref, out_ref,
                 s_ref, bx_ref, rowmax_ref):
    sig = jax.nn.sigmoid(logits_ref[...])
    s_ref[...] = sig
    rowmax_ref[...] = jnp.max(sig, axis=2)

    b = boxes_ref[...]
    xy = b[..., :2] - b[..., 2:] * 0.5
    bx_ref[...] = jnp.concatenate([xy, b[..., 2:]], axis=2) * size4_ref[0][None, None, :]

    iota_q = jax.lax.broadcasted_iota(jnp.int32, (_BLK, _NUM_Q), 1)
    iota_c = jax.lax.broadcasted_iota(jnp.int32, (1, _NUM_C), 1)

    def step(k, carry):
        rm = rowmax_ref[...]
        m = jnp.max(rm, axis=1, keepdims=True)
        qsel = jnp.min(jnp.where(rm == m, iota_q, _NUM_Q), axis=1)
        rows = jnp.concatenate(
            [s_ref[i, pl.ds(qsel[i], 1), :] for i in range(_BLK)], axis=0)
        box16 = jnp.concatenate(
            [bx_ref[i, pl.ds(qsel[i], 1), :] for i in range(_BLK)], axis=0)
        ci = jnp.min(jnp.where(rows == m, iota_c, _NUM_C), axis=1,
                     keepdims=True)
        new_rows = jnp.where(iota_c == ci, -1.0, rows)
        for i in range(_BLK):
            s_ref[i, pl.ds(qsel[i], 1), :] = new_rows[i:i + 1, :]
        nm = jnp.max(new_rows, axis=1)
        rowmax_ref[...] = jnp.where(iota_q == qsel[:, None], nm[:, None], rm)
        rows6 = jnp.concatenate([ci.astype(jnp.float32), m, box16], axis=1)
        rows6 = jnp.where(m > 0.0, rows6, jnp.zeros_like(rows6))
        out_ref[:, pl.ds(k, 1), :] = rows6[:, None, :]
        return carry

    jax.lax.fori_loop(0, _NUM_TOP, step, 0)


@jax.jit
def kernel(logits, boxes, original_sizes):
    n = logits.shape[0]
    osz = original_sizes[0].astype(jnp.float32)
    size4 = jnp.stack([osz[1], osz[0], osz[1], osz[0]])[None, :]

    grid = (n // _BLK,)
    out = pl.pallas_call(
        _detr_kernel,
        grid=grid,
        in_specs=[
            pl.BlockSpec((_BLK, _NUM_Q, _NUM_C), lambda b: (b, 0, 0)),
            pl.BlockSpec((_BLK, _NUM_Q, 4), lambda b: (b, 0, 0)),
            pl.BlockSpec((1, 4), lambda b: (0, 0)),
        ],
        out_specs=pl.BlockSpec((_BLK, _NUM_TOP, 6), lambda b: (b, 0, 0)),
        out_shape=jax.ShapeDtypeStruct((n, _NUM_TOP, 6), jnp.float32),
        scratch_shapes=[
            pltpu.VMEM((_BLK, _NUM_Q, _NUM_C), jnp.float32),
            pltpu.VMEM((_BLK, _NUM_Q, 4), jnp.float32),
            pltpu.VMEM((_BLK, _NUM_Q), jnp.float32),
        ],
        compiler_params=pltpu.CompilerParams(
            dimension_semantics=("parallel",)),
    )(logits, boxes, size4)
    return out
